# Initial kernel scaffold; baseline (speedup 1.0000x reference)
import jax
import jax.numpy as jnp
from jax.experimental import pallas as pl


def kernel(x, edge_index, mol_batch, target_x, target_edge_index, target_batch, params):
    raise NotImplementedError("write your pallas kernel here")


# speedup vs baseline: 3.1076x; 3.1076x over previous
import functools

import jax
import jax.numpy as jnp
from jax import lax
from jax.experimental import pallas as pl
from jax.experimental.pallas import tpu as pltpu
from jax.experimental.pallas import tpu_sc as plsc

f32 = jnp.float32
i32 = jnp.int32

N = 10000
NG = 256
E = 160000

N_PAD = 10112
RPT = N_PAD // 16
DUMP = 10008
NW = 32
EPW = 5120
NB = EPW // 128
E_PAD = NW * EPW


def _sc_mesh():
    return plsc.VectorSubcoreMesh(core_axis_name="c", subcore_axis_name="s",
                                  num_cores=2, num_subcores=16)


def _sc_scatter_fn(nchunk, fc):
    out_type = [jax.ShapeDtypeStruct((nchunk, 2 * N_PAD, fc), f32)]
    scratch = [
        pltpu.VMEM((NB, 128), i32),
        pltpu.VMEM((NB, 128), i32),
        pltpu.VMEM((128, fc), f32),
        pltpu.VMEM((128, fc), f32),
        pltpu.VMEM_SHARED((N_PAD, fc), f32),
        pltpu.SemaphoreType.DMA,
        pltpu.SemaphoreType.DMA,
    ]

    @functools.partial(pl.kernel, mesh=_sc_mesh(), out_type=out_type,
                       scratch_types=scratch)
    def k(src_hbm, dst_hbm, zeros_hbm, *rest):
        hs = rest[:nchunk]
        out = rest[nchunk]
        srcv, dstv, r0, r1, shared, s0, s1 = rest[nchunk + 1:]
        c = lax.axis_index("c")
        s = lax.axis_index("s")
        w = c * 16 + s
        pltpu.sync_copy(src_hbm.at[w], srcv)
        pltpu.sync_copy(dst_hbm.at[w], dstv)
        rows = (r0, r1)
        sems = (s0, s1)
        for ci in range(nchunk):
            pltpu.sync_copy(zeros_hbm, shared.at[pl.ds(s * RPT, RPT)])
            plsc.subcore_barrier()
            cp = pltpu.async_copy(hs[ci].at[srcv.at[0]], rows[0], sems[0])
            for b in range(NB):
                nb = b + 1
                if nb < NB:
                    cp_next = pltpu.async_copy(hs[ci].at[srcv.at[nb]],
                                               rows[nb % 2], sems[nb % 2])
                cp.wait()
                pltpu.sync_copy(rows[b % 2], shared.at[dstv.at[b]], add=True)
                if nb < NB:
                    cp = cp_next
            plsc.subcore_barrier()
            pltpu.sync_copy(shared.at[pl.ds(s * RPT, RPT)],
                            out.at[ci, pl.ds(c * N_PAD + s * RPT, RPT)])
            plsc.subcore_barrier()

    return k


def _sc_degree_fn():
    out_type = [jax.ShapeDtypeStruct((2 * N_PAD, 128), f32)]
    scratch = [
        pltpu.VMEM((NB, 128), i32),
        pltpu.VMEM((128, 128), f32),
        pltpu.VMEM_SHARED((N_PAD, 128), f32),
    ]

    @functools.partial(pl.kernel, mesh=_sc_mesh(), out_type=out_type,
                       scratch_types=scratch)
    def k(dst_hbm, zeros_hbm, ones_hbm, out, dstv, rows, shared):
        c = lax.axis_index("c")
        s = lax.axis_index("s")
        w = c * 16 + s
        pltpu.sync_copy(dst_hbm.at[w], dstv)
        pltpu.sync_copy(ones_hbm, rows)
        pltpu.sync_copy(zeros_hbm, shared.at[pl.ds(s * RPT, RPT)])
        plsc.subcore_barrier()
        for b in range(NB):
            pltpu.sync_copy(rows, shared.at[dstv.at[b]], add=True)
        plsc.subcore_barrier()
        pltpu.sync_copy(shared.at[pl.ds(s * RPT, RPT)],
                        out.at[pl.ds(c * N_PAD + s * RPT, RPT)])

    return k


def _dot(a, b):
    return jnp.dot(a, b, preferred_element_type=f32)


def _dotT(a, b):
    return lax.dot_general(a, b, dimension_numbers=(((0,), (0,)), ((), ())),
                           preferred_element_type=f32)


def _split(a):
    ai = jax.lax.bitcast_convert_type(a, jnp.uint32)
    ah = jax.lax.bitcast_convert_type(ai & jnp.uint32(0xFFFF0000), f32)
    return ah, a - ah


def _fdot(a, b):
    return _dot(a, b)


def _mdot(m, x):
    b16 = jnp.bfloat16
    xh, xr = _split(x)
    xm, xl = _split(xr)
    return (_dot(m, xh.astype(b16)) + _dot(m, xm.astype(b16))
            + _dot(m, xl.astype(b16)))


def _mdotT(m, x):
    b16 = jnp.bfloat16
    xh, xr = _split(x)
    xm, xl = _split(xr)
    return (_dotT(m, xh.astype(b16)) + _dotT(m, xm.astype(b16))
            + _dotT(m, xl.astype(b16)))


def _onehot(batch_pad):
    iota = lax.broadcasted_iota(i32, (NG, N_PAD), 0)
    return (batch_pad[None, :] == iota).astype(jnp.bfloat16)


def _cnt(M):
    ones = jnp.ones((N_PAD, 8), jnp.bfloat16)
    return jnp.maximum(_dot(M, ones)[:, :1], 1.0)


def _tc_call(body, out_shapes, *args, grid=None, in_specs=None, out_specs=None):
    kw = {}
    if grid is not None:
        kw = dict(grid=grid, in_specs=in_specs, out_specs=out_specs)
    return pl.pallas_call(body, out_shape=out_shapes, **kw)(*args)


def _t0(x_pad, w1p, degm):
    f1p = w1p.shape[1]

    def body(x_ref, w_ref, deg_ref, hs_ref, dinv_ref):
        deg = deg_ref[...][0:N_PAD, 0] + deg_ref[...][N_PAD:2 * N_PAD, 0]
        dinv = jnp.where(deg > 0, 1.0 / jnp.sqrt(deg), 0.0)[:, None]
        hs_ref[...] = _fdot(x_ref[...], w_ref[...]) * dinv
        dinv_ref[...] = dinv

    return _tc_call(body,
                    [jax.ShapeDtypeStruct((N_PAD, f1p), f32),
                     jax.ShapeDtypeStruct((N_PAD, 1), f32)],
                    x_pad, w1p, degm)


def _t_stats(acc, dinv, b, a, batch_pad):
    c = acc.shape[0]
    fp = c * 128

    def body(acc_ref, dinv_ref, b_ref, a_ref, bp_ref, xv_ref, mean_ref,
             var_ref):
        a3 = acc_ref[...][0]
        xv = ((a3[0:N_PAD] + a3[N_PAD:2 * N_PAD]) * dinv_ref[...]
              + b_ref[...][None, :])
        M = _onehot(bp_ref[...])
        cnt = _cnt(M)
        xv_ref[...] = xv
        mean = _mdot(M, xv) / cnt
        mean_ref[...] = mean
        sub = xv - a_ref[...][None, :] * _mdotT(M, mean)
        var_ref[...] = _mdot(M, sub * sub) / cnt

    return _tc_call(
        body,
        [jax.ShapeDtypeStruct((N_PAD, fp), f32),
         jax.ShapeDtypeStruct((NG, fp), f32),
         jax.ShapeDtypeStruct((NG, fp), f32)],
        acc, dinv, b, a, batch_pad,
        grid=(c,),
        in_specs=[pl.BlockSpec((1, 2 * N_PAD, 128), lambda i: (i, 0, 0)),
                  pl.BlockSpec((N_PAD, 1), lambda i: (0, 0)),
                  pl.BlockSpec((128,), lambda i: (i,)),
                  pl.BlockSpec((128,), lambda i: (i,)),
                  pl.BlockSpec((N_PAD,), lambda i: (0,))],
        out_specs=[pl.BlockSpec((N_PAD, 128), lambda i: (0, i)),
                   pl.BlockSpec((NG, 128), lambda i: (0, i)),
                   pl.BlockSpec((NG, 128), lambda i: (0, i))])


def _norm_y(xv_ref, mean_ref, var_ref, g_ref, bt_ref, a_ref, bp_ref):
    M = _onehot(bp_ref[...])
    meanb = _mdotT(M, mean_ref[...])
    varb = _mdotT(M, var_ref[...])
    sub = xv_ref[...] - a_ref[...][None, :] * meanb
    y = g_ref[...][None, :] * sub / jnp.sqrt(varb + 1e-5) + bt_ref[...][None, :]
    return jnp.maximum(y, 0.0), M


def _t_apply_mm(xv, mean, ex2, g, bt, a, wnext, dinv, batch_pad):
    c = xv.shape[1] // 128
    fnp = wnext.shape[1]

    co = fnp // 128

    def body(xv_ref, mean_ref, ex2_ref, g_ref, bt_ref, a_ref, w_ref,
             dinv_ref, bp_ref, hs_ref):
        i = pl.program_id(1)
        y, _ = _norm_y(xv_ref, mean_ref, ex2_ref, g_ref, bt_ref, a_ref, bp_ref)
        contrib = _fdot(y, w_ref[...]) * dinv_ref[...]

        @pl.when(i == 0)
        def _():
            hs_ref[...] = contrib

        @pl.when(i != 0)
        def _():
            hs_ref[...] += contrib

    return _tc_call(
        body, jax.ShapeDtypeStruct((N_PAD, fnp), f32),
        xv, mean, ex2, g, bt, a, wnext, dinv, batch_pad,
        grid=(co, c),
        in_specs=[pl.BlockSpec((N_PAD, 128), lambda j, i: (0, i)),
                  pl.BlockSpec((NG, 128), lambda j, i: (0, i)),
                  pl.BlockSpec((NG, 128), lambda j, i: (0, i)),
                  pl.BlockSpec((128,), lambda j, i: (i,)),
                  pl.BlockSpec((128,), lambda j, i: (i,)),
                  pl.BlockSpec((128,), lambda j, i: (i,)),
                  pl.BlockSpec((128, 128), lambda j, i: (i, j)),
                  pl.BlockSpec((N_PAD, 1), lambda j, i: (0, 0)),
                  pl.BlockSpec((N_PAD,), lambda j, i: (0,))],
        out_specs=pl.BlockSpec((N_PAD, 128), lambda j, i: (0, j)))


def _t_apply_pool(xv, mean, ex2, g, bt, a, batch_pad):
    c = xv.shape[1] // 128
    fp = c * 128

    def body(xv_ref, mean_ref, ex2_ref, g_ref, bt_ref, a_ref, bp_ref,
             out_ref):
        y, M = _norm_y(xv_ref, mean_ref, ex2_ref, g_ref, bt_ref, a_ref, bp_ref)
        out_ref[...] = _mdot(M, y) / _cnt(M)

    return _tc_call(
        body, jax.ShapeDtypeStruct((NG, fp), f32),
        xv, mean, ex2, g, bt, a, batch_pad,
        grid=(c,),
        in_specs=[pl.BlockSpec((N_PAD, 128), lambda i: (0, i)),
                  pl.BlockSpec((NG, 128), lambda i: (0, i)),
                  pl.BlockSpec((NG, 128), lambda i: (0, i)),
                  pl.BlockSpec((128,), lambda i: (i,)),
                  pl.BlockSpec((128,), lambda i: (i,)),
                  pl.BlockSpec((128,), lambda i: (i,)),
                  pl.BlockSpec((N_PAD,), lambda i: (0,))],
        out_specs=pl.BlockSpec((NG, 128), lambda i: (0, i)))


def _t_head(pm, pt, p):
    args = [pm, pt,
            p['Wg1p'], p['bg1'], p['bn1_g'], p['bn1_b'], p['Wg2'], p['bg2'],
            p['Wx1'], p['bx1'], p['bn2_g'], p['bn2_b'], p['Wx2'], p['bx2'],
            p['Wf1'], p['bf1'], p['bn3_g'], p['bn3_b'],
            p['Wf2'], p['bf2'], p['bn4_g'], p['bn4_b'],
            p['Wo'], p['bo']]

    def bnorm(x, g, b):
        m = jnp.mean(x, axis=0)
        v = jnp.mean((x - m[None, :]) ** 2, axis=0)
        return g[None, :] * (x - m[None, :]) / jnp.sqrt(v + 1e-5) + b[None, :]

    def body(pm_ref, pt_ref, wg1, bg1, bn1g, bn1b, wg2, bg2,
             wx1, bx1, bn2g, bn2b, wx2, bx2,
             wf1, bf1, bn3g, bn3b, wf2, bf2, bn4g, bn4b, wo, bo, out_ref):
        relu = lambda t: jnp.maximum(t, 0.0)
        h = relu(bnorm(_fdot(pm_ref[...], wg1[...]) + bg1[...][None, :],
                       bn1g[...], bn1b[...]))
        h = _fdot(h, wg2[...]) + bg2[...][None, :]
        t = relu(bnorm(_fdot(pt_ref[...], wx1[...]) + bx1[...][None, :],
                       bn2g[...], bn2b[...]))
        t = _fdot(t, wx2[...]) + bx2[...][None, :]
        xc = jnp.concatenate([h, t], axis=1)
        xc = relu(bnorm(_fdot(xc, wf1[...]) + bf1[...][None, :],
                        bn3g[...], bn3b[...]))
        xc = relu(bnorm(_fdot(xc, wf2[...]) + bf2[...][None, :],
                        bn4g[...], bn4b[...]))
        out_ref[...] = _fdot(xc, wo[...]) + bo[...][None, :]

    return _tc_call(body, jax.ShapeDtypeStruct((NG, 1), f32), *args)


def _stage_edges(ei):
    src = jnp.concatenate([ei[0], jnp.zeros((E_PAD - E,), i32)])
    dst = jnp.concatenate([ei[1], jnp.full((E_PAD - E,), DUMP, i32)])
    return src.reshape(NW, NB, 128), dst.reshape(NW, NB, 128)


def _padw(w, rpad, cpad):
    return jnp.pad(w, ((0, rpad - w.shape[0]), (0, cpad - w.shape[1])))


def _padv(v, n):
    return jnp.pad(v, (0, n - v.shape[0]))


def _branch(x, ei, batch, dims, wkeys, gkeys, params):
    p = params
    src3, dst3 = _stage_edges(ei)
    zeros128 = jnp.zeros((RPT, 128), f32)
    ones128 = jnp.ones((128, 128), f32)
    (degm,) = _sc_degree_fn()(dst3, zeros128, ones128)

    batch_pad = jnp.concatenate([batch, jnp.full((N_PAD - N,), NG, i32)])
    x_pad = jnp.pad(x, ((0, N_PAD - N), (0, 0)))

    (f1p, c1, fc1), (f2p, c2, fc2), (f3p, c3, fc3) = dims
    (w1k, b1k), (w2k, b2k), (w3k, b3k) = wkeys
    g1k, g2k, g3k = gkeys

    w1p = _padw(p[w1k], p[w1k].shape[0], f1p)
    w2p = _padw(p[w2k], f1p, f2p)
    w3p = _padw(p[w3k], f2p, f3p)

    def npad(prefix, fp):
        return (_padv(p[prefix + '_g'], fp), _padv(p[prefix + '_b'], fp),
                _padv(p[prefix + '_a'], fp))

    hs1, dinv = _t0(x_pad, w1p, degm)

    def conv(hs, nchunk, fc):
        zeros = jnp.zeros((RPT, fc), f32)
        chunks = [hs[:, ci * fc:(ci + 1) * fc] for ci in range(nchunk)]
        (acc,) = _sc_scatter_fn(nchunk, fc)(src3, dst3, zeros, *chunks)
        return acc

    acc1 = conv(hs1, c1, fc1)
    g1a = npad(g1k, f1p)
    g1, bt1, a1 = g1a
    xv1, m1, e1 = _t_stats(acc1, dinv, _padv(p[b1k], f1p), g1a[2], batch_pad)
    hs2 = _t_apply_mm(xv1, m1, e1, g1, bt1, a1, w2p, dinv, batch_pad)

    acc2 = conv(hs2, c2, fc2)
    g2a = npad(g2k, f2p)
    g2, bt2, a2 = g2a
    xv2, m2, e2 = _t_stats(acc2, dinv, _padv(p[b2k], f2p), g2a[2], batch_pad)
    hs3 = _t_apply_mm(xv2, m2, e2, g2, bt2, a2, w3p, dinv, batch_pad)

    acc3 = conv(hs3, c3, fc3)
    g3a = npad(g3k, f3p)
    g3, bt3, a3 = g3a
    xv3, m3, e3 = _t_stats(acc3, dinv, _padv(p[b3k], f3p), g3a[2], batch_pad)
    return _t_apply_pool(xv3, m3, e3, g3, bt3, a3, batch_pad)


def kernel(x, edge_index, mol_batch, target_x, target_edge_index,
           target_batch, params):
    p = dict(params)

    pooled_m = _branch(
        x, edge_index, mol_batch,
        dims=[(128, 1, 128), (256, 2, 128), (384, 3, 128)],
        wkeys=[('dW1', 'db1'), ('dW2', 'db2'), ('dW3', 'db3')],
        gkeys=['dg1', 'dg2', 'dg3'], params=p)

    pooled_t = _branch(
        target_x, target_edge_index, target_batch,
        dims=[(128, 1, 128), (256, 2, 128), (512, 4, 128)],
        wkeys=[('tW1', 'tb1'), ('tW2', 'tb2'), ('tW3', 'tb3')],
        gkeys=['tg1', 'tg2', 'tg3'], params=p)

    p['Wg1p'] = _padw(p['Wg1'], 384, 1024)
    return _t_head(pooled_m, pooled_t, p)
